# baseline (speedup 1.0000x reference)
import functools

import numpy as np
import jax
import jax.numpy as jnp
from jax import lax
from jax.experimental import pallas as pl
from jax.experimental.pallas import tpu as pltpu
from jax.experimental.pallas import tpu_sc as plsc

KMAX = 96
H = 128
BSLOT = 4 * H
JB = 8
NW = 32
LANE = 16
CPT = 16


def _sc_expand(xf, icat, fcat, kr, kc, kb, sri, srf):
    S = xf.shape[0]
    T = 16
    pch = CPT * KMAX
    bch = CPT * BSLOT
    mesh = plsc.VectorSubcoreMesh(core_axis_name="c", subcore_axis_name="s")

    @functools.partial(
        pl.kernel,
        out_type=(jax.ShapeDtypeStruct((NW * pch,), jnp.float32),
                  jax.ShapeDtypeStruct((NW * bch,), jnp.float32)),
        mesh=mesh,
        compiler_params=pltpu.CompilerParams(needs_layout_passes=False),
        scratch_types=[
            pltpu.VMEM((S,), jnp.float32),
            pltpu.VMEM((sri,), jnp.int32),
            pltpu.VMEM((srf,), jnp.float32),
            pltpu.VMEM((pch,), jnp.int32),
            pltpu.VMEM((pch,), jnp.int32),
            pltpu.VMEM((bch,), jnp.int32),
            pltpu.VMEM((pch,), jnp.float32),
            pltpu.VMEM((bch,), jnp.float32),
        ],
    )
    def gk(x_hbm, i_hbm, f_hbm, kr_hbm, kc_hbm, kb_hbm, op_hbm, ob_hbm,
           x_v, i_v, f_v, kr_v, kc_v, kb_v, op_v, ob_v):
        wid = lax.axis_index("s") * 2 + lax.axis_index("c")
        pltpu.sync_copy(x_hbm, x_v)
        pltpu.sync_copy(i_hbm.at[pl.ds(wid * sri, sri)], i_v)
        pltpu.sync_copy(f_hbm.at[pl.ds(wid * srf, srf)], f_v)
        pltpu.sync_copy(kr_hbm.at[pl.ds(wid * pch, pch)], kr_v)
        pltpu.sync_copy(kc_hbm.at[pl.ds(wid * pch, pch)], kc_v)
        pltpu.sync_copy(kb_hbm.at[pl.ds(wid * bch, bch)], kb_v)
        for i in range(pch // LANE):
            off = i * LANE
            rr = plsc.load_gather(i_v, [kr_v[pl.ds(off, LANE)]])
            cc = plsc.load_gather(i_v, [kc_v[pl.ds(off, LANE)]])
            op_v[pl.ds(off, LANE)] = plsc.load_gather(x_v, [rr * T + cc])
        for i in range(bch // LANE):
            off = i * LANE
            ob_v[pl.ds(off, LANE)] = plsc.load_gather(
                f_v, [kb_v[pl.ds(off, LANE)]])
        pltpu.sync_copy(op_v, op_hbm.at[pl.ds(wid * pch, pch)])
        pltpu.sync_copy(ob_v, ob_hbm.at[pl.ds(wid * bch, bch)])

    return gk(xf, icat, fcat, kr, kc, kb)


def _tc_group(nh, p4, hw, hb, wo4, bo4):
    NB = p4.shape[0]
    dn = (((1,), (1,)), ((), ()))

    def body(*refs):
        p_ref = refs[0]
        w_refs = refs[1:1 + nh]
        b_refs = refs[1 + nh:1 + 2 * nh]
        wo_ref, bo_ref, o_ref = refs[1 + 2 * nh:]
        rows = []
        for jj in range(JB):
            h = p_ref[0][jj:jj + 1, :]
            for l in range(nh):
                h = (lax.dot_general(h, w_refs[l][0][jj], dn)
                     + b_refs[l][0][jj:jj + 1, :])
                if l < nh - 1 or nh == 1:
                    h = jnp.maximum(h, 0.0)
            logit = jnp.sum(wo_ref[0][jj:jj + 1, :] * h, axis=1,
                            keepdims=True) + bo_ref[0][jj:jj + 1, 0:1]
            rows.append(jax.nn.sigmoid(logit))
        o_ref[0] = jnp.broadcast_to(jnp.concatenate(rows, axis=0), (JB, H))

    wspecs = [pl.BlockSpec((1, JB, H, KMAX), lambda b: (b, 0, 0, 0))]
    wspecs += [pl.BlockSpec((1, JB, H, H), lambda b: (b, 0, 0, 0))] * (nh - 1)
    vspec = pl.BlockSpec((1, JB, H), lambda b: (b, 0, 0))
    out = pl.pallas_call(
        body,
        grid=(NB,),
        in_specs=([pl.BlockSpec((1, JB, KMAX), lambda b: (b, 0, 0))]
                  + wspecs + [vspec] * nh + [vspec, vspec]),
        out_specs=vspec,
        out_shape=jax.ShapeDtypeStruct((NB, JB, H), jnp.float32),
    )(p4, *hw, *hb, wo4, bo4)
    return out[:, :, 0]


def _pad2(w, rows, cols):
    w = jnp.asarray(w, jnp.float32)
    return jnp.pad(w, ((0, rows - w.shape[0]), (0, cols - w.shape[1])))


def _rows_pad(x2d, rows):
    n = x2d.shape[0]
    if rows == n:
        return x2d
    return jnp.concatenate(
        [x2d, jnp.zeros((rows - n, x2d.shape[1]), x2d.dtype)])


def kernel(X, params, row_idx, col_idx):
    N, T = X.shape
    ncells = N * T

    nh_l = []
    for i in range(N):
        for t in range(T):
            nh_l.append(len(params[i][t]) - 1)
    nh_arr = np.asarray(nh_l)
    perm = np.argsort(nh_arr, kind="stable")
    inv = np.empty(ncells, np.int64)
    inv[perm] = np.arange(ncells)

    def cell_parts(cell):
        i, t = cell // T, cell % T
        return params[i][t], row_idx[i][t], col_idx[i][t]

    ntiles = NW
    i_ops, f_ops = [], []
    ioffs_r = np.zeros(ncells, np.int64)
    ioffs_c = np.zeros(ncells, np.int64)
    foffs_b = np.zeros((ncells, 3), np.int64)
    foffs_wo = np.zeros(ncells, np.int64)
    sri = 0
    srf = 0
    tiles_i, tiles_f = [], []
    for w in range(ntiles):
        icur, fcur = [], []
        ipos, fpos = 0, 8
        for cell in perm[w * CPT:(w + 1) * CPT]:
            ws, r, c = cell_parts(int(cell))
            nh = len(ws) - 1
            ioffs_r[cell] = ipos
            icur.append(r); ipos += len(r)
            ioffs_c[cell] = ipos
            icur.append(c); ipos += len(c)
            for l in range(nh):
                foffs_b[cell, l] = fpos
                fcur.append(ws[l][1]); fpos += ws[l][1].shape[0]
            foffs_wo[cell] = fpos
            fcur.append(ws[-1][0].reshape(-1)); fpos += ws[-1][0].shape[1]
            fcur.append(ws[-1][1]); fpos += 1
        tiles_i.append((icur, ipos))
        tiles_f.append((fcur, fpos))
        sri = max(sri, ipos)
        srf = max(srf, fpos)
    sri = -(-sri // 8) * 8
    srf = -(-srf // 8) * 8
    for w in range(ntiles):
        icur, ipos = tiles_i[w]
        fcur, fpos = tiles_f[w]
        i_ops.extend(icur)
        if ipos < sri:
            i_ops.append(np.zeros(sri - ipos, np.int32))
        f_ops.append(np.zeros(8, np.float32))
        f_ops.extend(fcur)
        if fpos < srf:
            f_ops.append(np.zeros(srf - fpos, np.float32))
    icat = jnp.concatenate([jnp.asarray(a).astype(jnp.int32).reshape(-1)
                            for a in i_ops])
    fcat = jnp.concatenate([jnp.asarray(a, jnp.float32).reshape(-1)
                            for a in f_ops])

    KR = np.zeros(ncells * KMAX, np.int32)
    KC = np.zeros(ncells * KMAX, np.int32)
    KB = np.zeros(ncells * BSLOT, np.int32)
    for pos in range(ncells):
        cell = int(perm[pos])
        ws, r, c = cell_parts(cell)
        nh = len(ws) - 1
        nr, nc = len(r), len(c)
        insz = nr * nc
        base = pos * KMAX
        for k in range(KMAX):
            KR[base + k] = ioffs_r[cell] + (k // nc if k < insz else 0)
            KC[base + k] = ioffs_c[cell] + (k % nc if k < insz else 0)
        bbase = pos * BSLOT
        for l in range(nh):
            hid = ws[l][1].shape[0]
            KB[bbase + l * H:bbase + l * H + hid] = \
                foffs_b[cell, l] + np.arange(hid)
        hid = ws[-1][0].shape[1]
        KB[bbase + 3 * H:bbase + 3 * H + hid] = \
            foffs_wo[cell] + np.arange(hid)

    outp, outb = _sc_expand(
        X.reshape(-1), icat, fcat,
        jnp.asarray(KR), jnp.asarray(KC), jnp.asarray(KB), sri, srf)
    patches = outp.reshape(ncells, KMAX)
    vblock = outb.reshape(ncells, BSLOT)

    bocat = jnp.concatenate(
        [cell_parts(int(cell))[0][-1][1] for cell in perm])

    outs = []
    start = 0
    for nh in (1, 2, 3):
        cells = [int(p) for p in perm[nh_arr[perm] == nh]]
        g = len(cells)
        if g == 0:
            continue
        gp = -(-g // JB) * JB
        NB = gp // JB

        p4 = _rows_pad(patches[start:start + g], gp).reshape(NB, JB, KMAX)
        hb4 = [_rows_pad(vblock[start:start + g, l * H:(l + 1) * H],
                         gp).reshape(NB, JB, H) for l in range(nh)]
        wo4 = _rows_pad(vblock[start:start + g, 3 * H:4 * H],
                        gp).reshape(NB, JB, H)
        bo4 = jnp.broadcast_to(
            jnp.pad(bocat[start:start + g], (0, gp - g))[:, None],
            (gp, H)).reshape(NB, JB, H)

        hw = [[] for _ in range(nh)]
        for cell in cells:
            ws = cell_parts(cell)[0]
            for l in range(nh):
                hw[l].append(_pad2(ws[l][0], H, KMAX if l == 0 else H))
        zW0 = jnp.zeros((H, KMAX), jnp.float32)
        zW = jnp.zeros((H, H), jnp.float32)
        for _ in range(gp - g):
            for l in range(nh):
                hw[l].append(zW0 if l == 0 else zW)
        hw4 = [jnp.stack(hw[l]).reshape(NB, JB, H, KMAX if l == 0 else H)
               for l in range(nh)]

        outs.append(_tc_group(nh, p4, hw4, hb4, wo4, bo4).reshape(-1)[:g])
        start += g

    out_all = jnp.concatenate(outs)
    return jnp.take(out_all, jnp.asarray(inv)).reshape(N, T)
